# Initial kernel scaffold; baseline (speedup 1.0000x reference)
import jax
import jax.numpy as jnp
from jax.experimental import pallas as pl


def kernel(X, sampled_nodes_per_layer, A_hat, W0, b0, W1, b1):
    raise NotImplementedError("write your pallas kernel here")


# speedup vs baseline: 4.0893x; 4.0893x over previous
import functools

import jax
import jax.numpy as jnp
from jax.experimental import pallas as pl
from jax.experimental.pallas import tpu as pltpu


_GROWS = 16


def _gather_rows_body(idx_ref, *refs):
    out_ref = refs[-1]
    for j in range(_GROWS):
        out_ref[0, j, :] = refs[j][0, 0, :]


def _gather_rows(table, idx, rows, cols):
    grid = rows // _GROWS
    table3 = table.reshape(table.shape[0], 1, cols)

    def mk_map(j):
        return lambda i, idx_ref: (idx_ref[i * _GROWS + j], 0, 0)

    grid_spec = pltpu.PrefetchScalarGridSpec(
        num_scalar_prefetch=1,
        grid=(grid,),
        in_specs=[pl.BlockSpec((1, 1, cols), mk_map(j)) for j in range(_GROWS)],
        out_specs=pl.BlockSpec((1, _GROWS, cols), lambda i, idx_ref: (i, 0, 0)),
    )
    out = pl.pallas_call(
        _gather_rows_body,
        grid_spec=grid_spec,
        out_shape=jax.ShapeDtypeStruct((grid, _GROWS, cols), table.dtype),
    )(idx, *([table3] * _GROWS))
    return out.reshape(rows, cols)


def _linear_scatter_body(n_nodes, scale, idx_ref, g_ref, w_ref, b_ref, out_ref, h_ref):
    h = jnp.dot(g_ref[...], w_ref[...], preferred_element_type=jnp.float32)
    h_ref[...] = jnp.maximum(h + b_ref[...], 0.0) * scale
    out_ref[...] = jnp.zeros_like(out_ref)
    s = g_ref.shape[0]

    def body(i, _):
        r = idx_ref[i]
        out_ref[pl.ds(r, 1), :] += h_ref[pl.ds(i, 1), :]
        return 0

    jax.lax.fori_loop(0, s, body, 0)


def _linear_scatter(idx, g, w, b, n_nodes, scale):
    dout = w.shape[1]
    grid_spec = pltpu.PrefetchScalarGridSpec(
        num_scalar_prefetch=1,
        grid=(1,),
        in_specs=[
            pl.BlockSpec(g.shape, lambda i, idx_ref: (0, 0)),
            pl.BlockSpec(w.shape, lambda i, idx_ref: (0, 0)),
            pl.BlockSpec((1, dout), lambda i, idx_ref: (0, 0)),
        ],
        out_specs=pl.BlockSpec((n_nodes, dout), lambda i, idx_ref: (0, 0)),
        scratch_shapes=[pltpu.VMEM((g.shape[0], dout), jnp.float32)],
    )
    return pl.pallas_call(
        functools.partial(_linear_scatter_body, n_nodes, scale),
        grid_spec=grid_spec,
        out_shape=jax.ShapeDtypeStruct((n_nodes, dout), jnp.float32),
    )(idx, g, w, b.reshape(1, dout))


def _mid_body(scale, idx_ref, a_ref, s0_ref, w1_ref, b1_ref, out_ref, h_ref):
    i = pl.program_id(0)
    agg = jnp.dot(a_ref[...], s0_ref[...], preferred_element_type=jnp.float32)
    h1 = jnp.maximum(agg, 0.0)
    h1 = jnp.dot(h1, w1_ref[...], preferred_element_type=jnp.float32)
    h_ref[...] = jnp.maximum(h1 + b1_ref[...], 0.0) * scale

    @pl.when(i == 0)
    def _():
        out_ref[...] = jnp.zeros_like(out_ref)

    blk = a_ref.shape[0]

    def body(j, _):
        r = idx_ref[i * blk + j]
        out_ref[pl.ds(r, 1), :] += h_ref[pl.ds(j, 1), :]
        return 0

    jax.lax.fori_loop(0, blk, body, 0)


def _mid_layer(idx, g1, scat0, w1, b1, n_nodes, scale, blk=256):
    s, k = g1.shape
    dout = w1.shape[1]
    grid_spec = pltpu.PrefetchScalarGridSpec(
        num_scalar_prefetch=1,
        grid=(s // blk,),
        in_specs=[
            pl.BlockSpec((blk, k), lambda i, idx_ref: (i, 0)),
            pl.BlockSpec(scat0.shape, lambda i, idx_ref: (0, 0)),
            pl.BlockSpec(w1.shape, lambda i, idx_ref: (0, 0)),
            pl.BlockSpec((1, dout), lambda i, idx_ref: (0, 0)),
        ],
        out_specs=pl.BlockSpec((n_nodes, dout), lambda i, idx_ref: (0, 0)),
        scratch_shapes=[pltpu.VMEM((blk, dout), jnp.float32)],
    )
    return pl.pallas_call(
        functools.partial(_mid_body, scale),
        grid_spec=grid_spec,
        out_shape=jax.ShapeDtypeStruct((n_nodes, dout), jnp.float32),
    )(idx, g1, scat0, w1, b1.reshape(1, dout))


def _dense_body(a_ref, s_ref, out_ref):
    out_ref[...] = jnp.dot(a_ref[...], s_ref[...], preferred_element_type=jnp.float32)


def _dense_matmul(a, scat, blk=256):
    n, k = a.shape
    dout = scat.shape[1]
    grid = pl.cdiv(n, blk)
    return pl.pallas_call(
        _dense_body,
        grid=(grid,),
        in_specs=[
            pl.BlockSpec((blk, k), lambda i: (i, 0)),
            pl.BlockSpec((k, dout), lambda i: (0, 0)),
        ],
        out_specs=pl.BlockSpec((blk, dout), lambda i: (i, 0)),
        out_shape=jax.ShapeDtypeStruct((n, dout), jnp.float32),
    )(a, scat)


def kernel(X, sampled_nodes_per_layer, A_hat, W0, b0, W1, b1):
    n, din = X.shape
    s = sampled_nodes_per_layer.shape[1]
    scale = float(n) / float(s)
    s0 = sampled_nodes_per_layer[0]
    s1 = sampled_nodes_per_layer[1]

    g0 = _gather_rows(X, s0, s, din)
    scat0 = _linear_scatter(s0, g0, W0, b0, n, scale)
    g1 = _gather_rows(A_hat, s1, s, n)
    scat1 = _mid_layer(s1, g1, scat0, W1, b1, n, scale)
    return _dense_matmul(A_hat, scat1)
